# baseline (speedup 1.0000x reference)
import functools

import jax
import jax.numpy as jnp
from jax import lax
from jax.experimental import pallas as pl
from jax.experimental.pallas import tpu as pltpu
from jax.experimental.pallas import tpu_sc as plsc

_NUM_CORES = 2
_NUM_SUBCORES = 16
_LANES = 16
_NUM_WORKERS = _NUM_CORES * _NUM_SUBCORES

_HBLK = 4
_NBUF = 2


@functools.lru_cache(maxsize=None)
def _build_sc_lookup(batch: int, hist: int, emb_dim: int, table_rows: int):
    assert emb_dim == _LANES
    bblk = batch // _NUM_WORKERS
    assert bblk * _NUM_WORKERS == batch and bblk % 128 == 0
    assert hist % (_HBLK * _NBUF) == 0
    n_pairs = hist // (_HBLK * _NBUF)

    mesh = plsc.VectorSubcoreMesh(core_axis_name="c", subcore_axis_name="s")

    @functools.partial(
        pl.kernel,
        out_type=jax.ShapeDtypeStruct((hist, emb_dim, batch), jnp.float32),
        mesh=mesh,
        compiler_params=pltpu.CompilerParams(
            needs_layout_passes=False, use_tc_tiling_on_sc=True
        ),
        scratch_types=[
            pltpu.VMEM((table_rows * emb_dim,), jnp.float32),
            pltpu.VMEM((_HBLK, bblk), jnp.int32),
            pltpu.VMEM((_HBLK, bblk), jnp.int32),
            pltpu.VMEM((_HBLK, emb_dim, bblk), jnp.float32),
            pltpu.VMEM((_HBLK, emb_dim, bblk), jnp.float32),
            pltpu.SemaphoreType.DMA,
            pltpu.SemaphoreType.DMA,
            pltpu.SemaphoreType.DMA,
            pltpu.SemaphoreType.DMA,
        ],
    )
    def emb_lookup(
        idxT_hbm, wt_hbm, x_hbm,
        wt_v, idx_v0, idx_v1, out_v0, out_v1, si0, si1, so0, so1,
    ):
        wid = lax.axis_index("s") * _NUM_CORES + lax.axis_index("c")
        b0 = wid * bblk
        idx_bufs = (idx_v0, idx_v1)
        out_bufs = (out_v0, out_v1)
        sin = (si0, si1)
        sout = (so0, so1)

        for par in range(_NBUF):
            pltpu.async_copy(
                idxT_hbm.at[pl.ds(par * _HBLK, _HBLK), pl.ds(b0, bblk)],
                idx_bufs[par],
                sin[par],
            )

        pltpu.sync_copy(wt_hbm, wt_v)
        lane_row = (
            jnp.minimum(lax.iota(jnp.int32, _LANES), table_rows - 1) * emb_dim
        )
        wcols = [plsc.load_gather(wt_v, [lane_row + e]) for e in range(emb_dim)]

        def chunk_body(hp, carry):
            for par in range(_NBUF):
                h0 = (hp * _NBUF + par) * _HBLK
                in_win = idxT_hbm.at[pl.ds(h0, _HBLK), pl.ds(b0, bblk)]
                out_win = x_hbm.at[pl.ds(h0, _HBLK), :, pl.ds(b0, bblk)]

                @pl.when(hp >= 1)
                def _wait_out():
                    pltpu.make_async_copy(
                        out_bufs[par], out_win, sout[par]
                    ).wait()

                pltpu.make_async_copy(in_win, idx_bufs[par], sin[par]).wait()

                for h in range(_HBLK):
                    def g_body(g, c2, h=h, par=par):
                        idx_vec = idx_bufs[par][h, pl.ds(g * _LANES, _LANES)]
                        for e in range(emb_dim):
                            vals = wcols[e].at[idx_vec].get(
                                mode="promise_in_bounds"
                            )
                            out_bufs[par][h, e, pl.ds(g * _LANES, _LANES)] = (
                                vals
                            )
                        return c2

                    lax.fori_loop(0, bblk // _LANES, g_body, 0, unroll=False)

                pltpu.async_copy(out_bufs[par], out_win, sout[par])

                @pl.when(hp + 1 < n_pairs)
                def _prefetch():
                    pltpu.async_copy(
                        idxT_hbm.at[
                            pl.ds(h0 + _NBUF * _HBLK, _HBLK), pl.ds(b0, bblk)
                        ],
                        idx_bufs[par],
                        sin[par],
                    )

            return carry

        lax.fori_loop(0, n_pairs, chunk_body, 0, unroll=False)

        for par in range(_NBUF):
            h0 = ((n_pairs - 1) * _NBUF + par) * _HBLK
            pltpu.make_async_copy(
                out_bufs[par],
                x_hbm.at[pl.ds(h0, _HBLK), :, pl.ds(b0, bblk)],
                sout[par],
            ).wait()

    return emb_lookup


def kernel(stage_idx, emb_weight):
    batch, hist = stage_idx.shape
    table_rows, emb_dim = emb_weight.shape
    idxT = stage_idx.T.astype(jnp.int32)
    wt = emb_weight.astype(jnp.float32).reshape(-1)
    fn = _build_sc_lookup(batch, hist, emb_dim, table_rows)
    x = fn(idxT, wt)
    return jnp.transpose(x, (2, 0, 1))
